# Initial kernel scaffold; baseline (speedup 1.0000x reference)
import jax
import jax.numpy as jnp
from jax.experimental import pallas as pl


def kernel(node_features, weight, root, bias, edge_weights, edge_index, edge_type, ent_user_ids, ent_item_ids, aspect_ent_ids):
    raise NotImplementedError("write your pallas kernel here")


# speedup vs baseline: 1.3642x; 1.3642x over previous
import functools

import jax
import jax.numpy as jnp
from jax import lax
from jax.experimental import pallas as pl
from jax.experimental.pallas import tpu as pltpu
from jax.experimental.pallas import tpu_sc as plsc

N_NODES = 10002
D = 128
R = 16
NB = 8
DB = 16
E = 320000
BATCH = 1024
N_ASPECT = 5

NP = 10240
NRP = NP * R
CR = NRP // 128
NC = 2
NS = 16
NW = NC * NS
EP = 327680
EPW = EP // NW
K = 128
TN = 512

_mesh = plsc.VectorSubcoreMesh(core_axis_name="c", subcore_axis_name="s")


def _h_body(x_ref, w_ref, out_ref):
    out_ref[0] = jnp.dot(x_ref[...], w_ref[0],
                         preferred_element_type=jnp.float32)


def _compute_h(x_pad, wd):
    return pl.pallas_call(
        _h_body,
        grid=(R, NP // TN),
        in_specs=[
            pl.BlockSpec((TN, D), lambda r, j: (j, 0)),
            pl.BlockSpec((1, D, D), lambda r, j: (r, 0, 0)),
        ],
        out_specs=pl.BlockSpec((1, TN, D), lambda r, j: (r, j, 0)),
        out_shape=jax.ShapeDtypeStruct((R, NP, D), jnp.float32),
    )(x_pad, wd)


@functools.partial(
    pl.kernel,
    mesh=_mesh,
    out_type=(
        jax.ShapeDtypeStruct((NC, CR, 128), jnp.float32),
        jax.ShapeDtypeStruct((EP,), jnp.int32),
        jax.ShapeDtypeStruct((EP,), jnp.int32),
    ),
    scratch_types=[
        pltpu.VMEM((K,), jnp.int32),
        pltpu.VMEM((K,), jnp.int32),
        pltpu.VMEM((K,), jnp.int32),
        pltpu.VMEM((K,), jnp.int32),
        pltpu.VMEM((K,), jnp.int32),
        pltpu.VMEM((K,), jnp.int32),
        pltpu.VMEM((K,), jnp.int32),
        pltpu.VMEM((K, 128), jnp.float32),
        pltpu.VMEM_SHARED((CR, 128), jnp.float32),
    ],
)
def _sc_counts(agg_hbm, et_hbm, mn_hbm, zeros_hbm,
               counts_out, seg_out, ge_out,
               a_v, e_v, m_v, seg_v, ge_v, hi_v, lo_v, oh_v, cnt_sh):
    cid = lax.axis_index("c")
    sid = lax.axis_index("s")
    wid = sid * NC + cid
    zrows = CR // NS
    pltpu.sync_copy(zeros_hbm, cnt_sh.at[pl.ds(sid * zrows, zrows)])
    plsc.subcore_barrier()

    def chunk(k, _):
        base = wid * EPW + k * K
        pltpu.sync_copy(agg_hbm.at[pl.ds(base, K)], a_v)
        pltpu.sync_copy(et_hbm.at[pl.ds(base, K)], e_v)
        pltpu.sync_copy(mn_hbm.at[pl.ds(base, K)], m_v)
        for i in range(K // 16):
            sl = pl.ds(i * 16, 16)
            et = e_v[sl]
            seg = a_v[sl] * R + et
            seg_v[sl] = seg
            ge_v[sl] = et * NP + m_v[sl]
            hi_v[sl] = seg >> 7
            lo_v[sl] = seg & 127
        pltpu.sync_copy(seg_v, seg_out.at[pl.ds(base, K)])
        pltpu.sync_copy(ge_v, ge_out.at[pl.ds(base, K)])

        lane = jnp.arange(16, dtype=jnp.int32)

        def ohg(g, _):
            sl = pl.ds(g * 16, 16)
            lo16 = lo_v[sl]
            for l in range(16):
                e = g * 16 + l
                col = lo16[l]
                for j in range(8):
                    val = jnp.where(lane + (16 * j) == col, 1.0, 0.0)
                    oh_v[e, pl.ds(j * 16, 16)] = val
            return 0

        lax.fori_loop(0, K // 16, ohg, 0)
        pltpu.sync_copy(oh_v, cnt_sh.at[hi_v], add=True)
        return 0

    lax.fori_loop(0, EPW // K, chunk, 0)
    plsc.subcore_barrier()
    pltpu.sync_copy(cnt_sh.at[pl.ds(sid * zrows, zrows)],
                    counts_out.at[cid, pl.ds(sid * zrows, zrows)])


def _inv_body(c_ref, out_ref):
    out_ref[...] = 1.0 / jnp.maximum(c_ref[0] + c_ref[1], 1.0)


def _compute_inv(counts):
    return pl.pallas_call(
        _inv_body,
        grid=(CR // 128,),
        in_specs=[pl.BlockSpec((NC, 128, 128), lambda j: (0, j, 0))],
        out_specs=pl.BlockSpec((128, 128), lambda j: (j, 0)),
        out_shape=jax.ShapeDtypeStruct((CR, 128), jnp.float32),
    )(counts)


@functools.partial(
    pl.kernel,
    mesh=_mesh,
    out_type=jax.ShapeDtypeStruct((NC, NP, D), jnp.float32),
    scratch_types=[
        pltpu.VMEM((K,), jnp.int32),
        pltpu.VMEM((K,), jnp.int32),
        pltpu.VMEM((K,), jnp.int32),
        pltpu.VMEM((K,), jnp.int32),
        pltpu.VMEM((K,), jnp.float32),
        pltpu.VMEM((K, 128), jnp.float32),
        pltpu.VMEM((K, D), jnp.float32),
        pltpu.VMEM((32,), jnp.float32),
        pltpu.VMEM_SHARED((NP, D), jnp.float32),
        pltpu.SemaphoreType.DMA,
        pltpu.SemaphoreType.DMA,
    ],
)
def _sc_edges(seg_hbm, agg_hbm, ge_hbm, ew_hbm, inv_hbm, h_hbm, zacc_hbm,
              acc_out,
              seg_v, agg_v, ge_v, hi_v, ew_v, inv_v, rows_v, fold_v, acc_sh,
              sem0, sem1):
    cid = lax.axis_index("c")
    sid = lax.axis_index("s")
    wid = sid * NC + cid
    zrows = NP // NS
    pltpu.sync_copy(zacc_hbm, acc_sh.at[pl.ds(sid * zrows, zrows)])
    plsc.subcore_barrier()
    lane = jnp.arange(16, dtype=jnp.int32)
    fold_v[pl.ds(16, 16)] = jnp.zeros((16,), jnp.float32)

    def chunk(k, _):
        base = wid * EPW + k * K
        pltpu.sync_copy(seg_hbm.at[pl.ds(base, K)], seg_v)
        pltpu.sync_copy(agg_hbm.at[pl.ds(base, K)], agg_v)
        pltpu.sync_copy(ge_hbm.at[pl.ds(base, K)], ge_v)
        pltpu.sync_copy(ew_hbm.at[pl.ds(base, K)], ew_v)
        for i in range(K // 16):
            sl = pl.ds(i * 16, 16)
            hi_v[sl] = seg_v[sl] >> 7
        pltpu.async_copy(inv_hbm.at[hi_v], inv_v, sem0).wait()
        pltpu.async_copy(h_hbm.at[ge_v], rows_v, sem1).wait()

        def sgroup(g, _):
            sl = pl.ds(g * 16, 16)
            lo16 = seg_v[sl] & 127
            ew16 = ew_v[sl]
            for l in range(16):
                e = g * 16 + l
                col = lo16[l]
                acc = jnp.zeros((16,), jnp.float32)
                for j in range(8):
                    cs = pl.ds(j * 16, 16)
                    acc = acc + jnp.where(lane + (16 * j) == col,
                                          inv_v[e, cs], 0.0)
                fold_v[pl.ds(0, 16)] = acc
                for step in (8, 4, 2, 1):
                    t = fold_v[pl.ds(0, 16)] + fold_v[pl.ds(step, 16)]
                    fold_v[pl.ds(0, 16)] = t
                s = t[0] * ew16[l]
                for j in range(D // 16):
                    cs = pl.ds(j * 16, 16)
                    rows_v[e, cs] = rows_v[e, cs] * s
            return 0

        lax.fori_loop(0, K // 16, sgroup, 0)
        pltpu.sync_copy(rows_v, acc_sh.at[agg_v], add=True)
        return 0

    lax.fori_loop(0, EPW // K, chunk, 0)
    plsc.subcore_barrier()
    pltpu.sync_copy(acc_sh.at[pl.ds(sid * zrows, zrows)],
                    acc_out.at[cid, pl.ds(sid * zrows, zrows)])


def _feat_body(acc_ref, x_ref, root_ref, bias_ref, out_ref):
    h = acc_ref[0] + acc_ref[1]
    h = h + jnp.dot(x_ref[...], root_ref[...],
                    preferred_element_type=jnp.float32)
    out_ref[...] = jnp.maximum(h + bias_ref[...], 0.0)


def _compute_features(acc, x_pad, root, bias2d):
    return pl.pallas_call(
        _feat_body,
        grid=(NP // TN,),
        in_specs=[
            pl.BlockSpec((NC, TN, D), lambda j: (0, j, 0)),
            pl.BlockSpec((TN, D), lambda j: (j, 0)),
            pl.BlockSpec((D, D), lambda j: (0, 0)),
            pl.BlockSpec((1, D), lambda j: (0, 0)),
        ],
        out_specs=pl.BlockSpec((TN, D), lambda j: (j, 0)),
        out_shape=jax.ShapeDtypeStruct((NP, D), jnp.float32),
    )(acc, x_pad, root, bias2d)


NID = BATCH * (2 + N_ASPECT)
IDW = NID // NW
IDC = 112


@functools.partial(
    pl.kernel,
    mesh=_mesh,
    out_type=jax.ShapeDtypeStruct((NID, D), jnp.float32),
    scratch_types=[
        pltpu.VMEM((IDC,), jnp.int32),
        pltpu.VMEM((IDC, D), jnp.float32),
        pltpu.SemaphoreType.DMA,
    ],
)
def _sc_gather_out(feat_hbm, ids_hbm, out_hbm, idx_v, rows_v, sem):
    cid = lax.axis_index("c")
    sid = lax.axis_index("s")
    wid = sid * NC + cid

    def chunk(k, _):
        base = wid * IDW + k * IDC
        pltpu.sync_copy(ids_hbm.at[pl.ds(base, IDC)], idx_v)
        pltpu.async_copy(feat_hbm.at[idx_v], rows_v, sem).wait()
        pltpu.sync_copy(rows_v, out_hbm.at[pl.ds(base, IDC)])
        return 0

    lax.fori_loop(0, IDW // IDC, chunk, 0)


def kernel(node_features, weight, root, bias, edge_weights, edge_index,
           edge_type, ent_user_ids, ent_item_ids, aspect_ent_ids):
    f32 = jnp.float32
    i32 = jnp.int32

    w5 = jnp.zeros((R, NB, DB, NB, DB), f32)
    bidx = jnp.arange(NB)
    w5 = w5.at[:, bidx, :, bidx, :].set(weight.transpose(1, 0, 2, 3))
    wd = w5.reshape(R, D, D)

    x_pad = jnp.zeros((NP, D), f32).at[:N_NODES].set(node_features)

    pad = EP - E
    agg = jnp.concatenate(
        [edge_index[0].astype(i32), jnp.full((pad,), NP - 1, i32)])
    mn = jnp.concatenate([edge_index[1].astype(i32), jnp.zeros((pad,), i32)])
    et = jnp.concatenate([edge_type.astype(i32), jnp.zeros((pad,), i32)])
    ew = jnp.concatenate([edge_weights[0].astype(f32), jnp.zeros((pad,), f32)])

    zeros_cnt = jnp.zeros((CR // NS, 128), f32)
    zeros_acc = jnp.zeros((NP // NS, D), f32)

    h = _compute_h(x_pad, wd).reshape(R * NP, D)
    counts, seg, ge = _sc_counts(agg, et, mn, zeros_cnt)
    inv = _compute_inv(counts)
    acc = _sc_edges(seg, agg, ge, ew, inv, h, zeros_acc)
    feats = _compute_features(acc, x_pad, root, bias.reshape(1, D))

    ids = jnp.concatenate([
        ent_user_ids.astype(i32),
        ent_item_ids.astype(i32),
        aspect_ent_ids.astype(i32).reshape(-1),
    ])
    g = _sc_gather_out(feats, ids)
    ent_user_rep = g[:BATCH][:, None, :]
    ent_item_rep = g[BATCH:2 * BATCH][:, None, :]
    ent_aspect_rep = g[2 * BATCH:].reshape(BATCH, N_ASPECT, D)
    return (ent_user_rep, ent_item_rep, ent_aspect_rep)


# speedup vs baseline: 1.7501x; 1.2829x over previous
import functools

import jax
import jax.numpy as jnp
from jax import lax
from jax.experimental import pallas as pl
from jax.experimental.pallas import tpu as pltpu
from jax.experimental.pallas import tpu_sc as plsc

N_NODES = 10002
D = 128
R = 16
NB = 8
DB = 16
E = 320000
BATCH = 1024
N_ASPECT = 5

NP = 10240
NRP = NP * R
CR = NRP // 128
NC = 2
NS = 16
NW = NC * NS
EP = 327680
EPW = EP // NW
K = 128
TN = 512

_mesh = plsc.VectorSubcoreMesh(core_axis_name="c", subcore_axis_name="s")


def _h_body(x_ref, w_ref, out_ref):
    out_ref[0] = jnp.dot(x_ref[...], w_ref[0],
                         preferred_element_type=jnp.float32)


def _compute_h(x_pad, wd):
    return pl.pallas_call(
        _h_body,
        grid=(R, NP // TN),
        in_specs=[
            pl.BlockSpec((TN, D), lambda r, j: (j, 0)),
            pl.BlockSpec((1, D, D), lambda r, j: (r, 0, 0)),
        ],
        out_specs=pl.BlockSpec((1, TN, D), lambda r, j: (r, j, 0)),
        out_shape=jax.ShapeDtypeStruct((R, NP, D), jnp.float32),
    )(x_pad, wd)


@functools.partial(
    pl.kernel,
    mesh=_mesh,
    out_type=(
        jax.ShapeDtypeStruct((NC, CR, 128), jnp.float32),
        jax.ShapeDtypeStruct((EP,), jnp.int32),
        jax.ShapeDtypeStruct((EP,), jnp.int32),
    ),
    scratch_types=[
        pltpu.VMEM((3, K), jnp.int32),
        pltpu.VMEM((K,), jnp.int32),
        pltpu.VMEM((K,), jnp.int32),
        pltpu.VMEM((K,), jnp.int32),
        pltpu.VMEM((K,), jnp.int32),
        pltpu.VMEM((K, 128), jnp.float32),
        pltpu.VMEM_SHARED((CR, 128), jnp.float32),
    ],
)
def _sc_counts(edata_hbm, zeros_hbm,
               counts_out, seg_out, ge_out,
               ed_v, seg_v, ge_v, hi_v, lo_v, oh_v, cnt_sh):
    cid = lax.axis_index("c")
    sid = lax.axis_index("s")
    wid = sid * NC + cid
    zrows = CR // NS
    pltpu.sync_copy(zeros_hbm, cnt_sh.at[pl.ds(sid * zrows, zrows)])
    plsc.subcore_barrier()

    def chunk(k, _):
        base = wid * EPW + k * K
        pltpu.sync_copy(edata_hbm.at[wid * (EPW // K) + k], ed_v)
        for i in range(K // 16):
            sl = pl.ds(i * 16, 16)
            et = ed_v[1, sl]
            seg = ed_v[0, sl] * R + et
            seg_v[sl] = seg
            ge_v[sl] = et * NP + ed_v[2, sl]
            hi_v[sl] = seg >> 7
            lo_v[sl] = seg & 127
        pltpu.sync_copy(seg_v, seg_out.at[pl.ds(base, K)])
        pltpu.sync_copy(ge_v, ge_out.at[pl.ds(base, K)])

        lane = jnp.arange(16, dtype=jnp.int32)

        def ohg(g, _):
            sl = pl.ds(g * 16, 16)
            lo16 = lo_v[sl]
            for l in range(16):
                e = g * 16 + l
                col = lo16[l]
                for j in range(8):
                    val = jnp.where(lane + (16 * j) == col, 1.0, 0.0)
                    oh_v[e, pl.ds(j * 16, 16)] = val
            return 0

        lax.fori_loop(0, K // 16, ohg, 0)
        pltpu.sync_copy(oh_v, cnt_sh.at[hi_v], add=True)
        return 0

    lax.fori_loop(0, EPW // K, chunk, 0)
    plsc.subcore_barrier()
    pltpu.sync_copy(cnt_sh.at[pl.ds(sid * zrows, zrows)],
                    counts_out.at[cid, pl.ds(sid * zrows, zrows)])


def _inv_body(c_ref, out_ref):
    out_ref[...] = 1.0 / jnp.maximum(c_ref[0] + c_ref[1], 1.0)


def _compute_inv(counts):
    return pl.pallas_call(
        _inv_body,
        grid=(CR // 128,),
        in_specs=[pl.BlockSpec((NC, 128, 128), lambda j: (0, j, 0))],
        out_specs=pl.BlockSpec((128, 128), lambda j: (j, 0)),
        out_shape=jax.ShapeDtypeStruct((CR, 128), jnp.float32),
    )(counts)


@functools.partial(
    pl.kernel,
    mesh=_mesh,
    out_type=jax.ShapeDtypeStruct((NC, NP, D), jnp.float32),
    scratch_types=[
        pltpu.VMEM((3, K), jnp.int32),
        pltpu.VMEM((K,), jnp.int32),
        pltpu.VMEM((K,), jnp.float32),
        pltpu.VMEM((K, 128), jnp.float32),
        pltpu.VMEM((K, D), jnp.float32),
        pltpu.VMEM((32,), jnp.float32),
        pltpu.VMEM_SHARED((NP, D), jnp.float32),
        pltpu.SemaphoreType.DMA,
        pltpu.SemaphoreType.DMA,
    ],
)
def _sc_edges(edata_hbm, ew_hbm, inv_hbm, h_hbm, zacc_hbm,
              acc_out,
              ed_v, hi_v, ew_v, inv_v, rows_v, fold_v, acc_sh,
              sem0, sem1):
    cid = lax.axis_index("c")
    sid = lax.axis_index("s")
    wid = sid * NC + cid
    zrows = NP // NS
    pltpu.sync_copy(zacc_hbm, acc_sh.at[pl.ds(sid * zrows, zrows)])
    plsc.subcore_barrier()
    lane = jnp.arange(16, dtype=jnp.int32)
    fold_v[pl.ds(16, 16)] = jnp.zeros((16,), jnp.float32)

    def chunk(k, _):
        base = wid * EPW + k * K
        pltpu.sync_copy(edata_hbm.at[wid * (EPW // K) + k], ed_v)
        pltpu.sync_copy(ew_hbm.at[pl.ds(base, K)], ew_v)
        for i in range(K // 16):
            sl = pl.ds(i * 16, 16)
            hi_v[sl] = ed_v[0, sl] >> 7
        c1 = pltpu.async_copy(inv_hbm.at[hi_v], inv_v, sem0)
        c2 = pltpu.async_copy(h_hbm.at[ed_v.at[2]], rows_v, sem1)
        c1.wait()
        c2.wait()

        def sgroup(g, _):
            sl = pl.ds(g * 16, 16)
            lo16 = ed_v[0, sl] & 127
            ew16 = ew_v[sl]
            for l in range(16):
                e = g * 16 + l
                col = lo16[l]
                acc = jnp.zeros((16,), jnp.float32)
                for j in range(8):
                    cs = pl.ds(j * 16, 16)
                    acc = acc + jnp.where(lane + (16 * j) == col,
                                          inv_v[e, cs], 0.0)
                fold_v[pl.ds(0, 16)] = acc
                for step in (8, 4, 2, 1):
                    t = fold_v[pl.ds(0, 16)] + fold_v[pl.ds(step, 16)]
                    fold_v[pl.ds(0, 16)] = t
                s = t[0] * ew16[l]
                for j in range(D // 16):
                    cs = pl.ds(j * 16, 16)
                    rows_v[e, cs] = rows_v[e, cs] * s
            return 0

        lax.fori_loop(0, K // 16, sgroup, 0)
        pltpu.sync_copy(rows_v, acc_sh.at[ed_v.at[1]], add=True)
        return 0

    lax.fori_loop(0, EPW // K, chunk, 0)
    plsc.subcore_barrier()
    pltpu.sync_copy(acc_sh.at[pl.ds(sid * zrows, zrows)],
                    acc_out.at[cid, pl.ds(sid * zrows, zrows)])


def _feat_body(acc_ref, x_ref, root_ref, bias_ref, out_ref):
    h = acc_ref[0] + acc_ref[1]
    h = h + jnp.dot(x_ref[...], root_ref[...],
                    preferred_element_type=jnp.float32)
    out_ref[...] = jnp.maximum(h + bias_ref[...], 0.0)


def _compute_features(acc, x_pad, root, bias2d):
    return pl.pallas_call(
        _feat_body,
        grid=(NP // TN,),
        in_specs=[
            pl.BlockSpec((NC, TN, D), lambda j: (0, j, 0)),
            pl.BlockSpec((TN, D), lambda j: (j, 0)),
            pl.BlockSpec((D, D), lambda j: (0, 0)),
            pl.BlockSpec((1, D), lambda j: (0, 0)),
        ],
        out_specs=pl.BlockSpec((TN, D), lambda j: (j, 0)),
        out_shape=jax.ShapeDtypeStruct((NP, D), jnp.float32),
    )(acc, x_pad, root, bias2d)


NID = BATCH * (2 + N_ASPECT)
IDW = NID // NW
IDC = 112


@functools.partial(
    pl.kernel,
    mesh=_mesh,
    out_type=jax.ShapeDtypeStruct((NID, D), jnp.float32),
    scratch_types=[
        pltpu.VMEM((IDC,), jnp.int32),
        pltpu.VMEM((IDC, D), jnp.float32),
        pltpu.SemaphoreType.DMA,
    ],
)
def _sc_gather_out(feat_hbm, ids_hbm, out_hbm, idx_v, rows_v, sem):
    cid = lax.axis_index("c")
    sid = lax.axis_index("s")
    wid = sid * NC + cid

    def chunk(k, _):
        base = wid * IDW + k * IDC
        pltpu.sync_copy(ids_hbm.at[pl.ds(base, IDC)], idx_v)
        pltpu.async_copy(feat_hbm.at[idx_v], rows_v, sem).wait()
        pltpu.sync_copy(rows_v, out_hbm.at[pl.ds(base, IDC)])
        return 0

    lax.fori_loop(0, IDW // IDC, chunk, 0)


def kernel(node_features, weight, root, bias, edge_weights, edge_index,
           edge_type, ent_user_ids, ent_item_ids, aspect_ent_ids):
    f32 = jnp.float32
    i32 = jnp.int32

    w5 = jnp.zeros((R, NB, DB, NB, DB), f32)
    bidx = jnp.arange(NB)
    w5 = w5.at[:, bidx, :, bidx, :].set(weight.transpose(1, 0, 2, 3))
    wd = w5.reshape(R, D, D)

    x_pad = jnp.zeros((NP, D), f32).at[:N_NODES].set(node_features)

    pad = EP - E
    agg = jnp.concatenate(
        [edge_index[0].astype(i32), jnp.full((pad,), NP - 1, i32)])
    mn = jnp.concatenate([edge_index[1].astype(i32), jnp.zeros((pad,), i32)])
    et = jnp.concatenate([edge_type.astype(i32), jnp.zeros((pad,), i32)])
    ew = jnp.concatenate([edge_weights[0].astype(f32), jnp.zeros((pad,), f32)])

    zeros_cnt = jnp.zeros((CR // NS, 128), f32)
    zeros_acc = jnp.zeros((NP // NS, D), f32)

    h = _compute_h(x_pad, wd).reshape(R * NP, D)
    edata_a = jnp.stack(
        [agg.reshape(EP // K, K), et.reshape(EP // K, K),
         mn.reshape(EP // K, K)], axis=1)
    counts, seg, ge = _sc_counts(edata_a, zeros_cnt)
    inv = _compute_inv(counts)
    edata_b = jnp.stack(
        [seg.reshape(EP // K, K), agg.reshape(EP // K, K),
         ge.reshape(EP // K, K)], axis=1)
    acc = _sc_edges(edata_b, ew, inv, h, zeros_acc)
    feats = _compute_features(acc, x_pad, root, bias.reshape(1, D))

    ids = jnp.concatenate([
        ent_user_ids.astype(i32),
        ent_item_ids.astype(i32),
        aspect_ent_ids.astype(i32).reshape(-1),
    ])
    g = _sc_gather_out(feats, ids)
    ent_user_rep = g[:BATCH][:, None, :]
    ent_item_rep = g[BATCH:2 * BATCH][:, None, :]
    ent_aspect_rep = g[2 * BATCH:].reshape(BATCH, N_ASPECT, D)
    return (ent_user_rep, ent_item_rep, ent_aspect_rep)


# speedup vs baseline: 1.7960x; 1.0262x over previous
import functools

import jax
import jax.numpy as jnp
from jax import lax
from jax.experimental import pallas as pl
from jax.experimental.pallas import tpu as pltpu
from jax.experimental.pallas import tpu_sc as plsc

N_NODES = 10002
D = 128
R = 16
NB = 8
DB = 16
E = 320000
BATCH = 1024
N_ASPECT = 5

NP = 10240
NRP = NP * R
CR = NRP // 128
NC = 2
NS = 16
NW = NC * NS
EP = 327680
EPW = EP // NW
K = 128
TN = 512

_mesh = plsc.VectorSubcoreMesh(core_axis_name="c", subcore_axis_name="s")


def _h_body(x_ref, w_ref, out_ref):
    out_ref[0] = jnp.dot(x_ref[...], w_ref[0],
                         preferred_element_type=jnp.float32)


def _compute_h(x_pad, wd):
    return pl.pallas_call(
        _h_body,
        grid=(R, NP // TN),
        in_specs=[
            pl.BlockSpec((TN, D), lambda r, j: (j, 0)),
            pl.BlockSpec((1, D, D), lambda r, j: (r, 0, 0)),
        ],
        out_specs=pl.BlockSpec((1, TN, D), lambda r, j: (r, j, 0)),
        out_shape=jax.ShapeDtypeStruct((R, NP, D), jnp.float32),
    )(x_pad, wd)


@functools.partial(
    pl.kernel,
    mesh=_mesh,
    out_type=(
        jax.ShapeDtypeStruct((NC, CR, 128), jnp.float32),
        jax.ShapeDtypeStruct((EP,), jnp.int32),
        jax.ShapeDtypeStruct((EP,), jnp.int32),
    ),
    scratch_types=[
        pltpu.VMEM((3, K), jnp.int32),
        pltpu.VMEM((K,), jnp.int32),
        pltpu.VMEM((K,), jnp.int32),
        pltpu.VMEM((K,), jnp.int32),
        pltpu.VMEM((K,), jnp.int32),
        pltpu.VMEM((K, 128), jnp.float32),
        pltpu.VMEM_SHARED((CR, 128), jnp.float32),
        pltpu.SemaphoreType.DMA,
        pltpu.SemaphoreType.DMA,
    ],
)
def _sc_counts(edata_hbm, zeros_hbm,
               counts_out, seg_out, ge_out,
               ed_v, seg_v, ge_v, hi_v, lo_v, oh_v, cnt_sh, sema, semb):
    cid = lax.axis_index("c")
    sid = lax.axis_index("s")
    wid = sid * NC + cid
    zrows = CR // NS
    pltpu.sync_copy(zeros_hbm, cnt_sh.at[pl.ds(sid * zrows, zrows)])
    plsc.subcore_barrier()

    def chunk(k, _):
        base = wid * EPW + k * K
        pltpu.sync_copy(edata_hbm.at[wid * (EPW // K) + k], ed_v)
        for i in range(K // 16):
            sl = pl.ds(i * 16, 16)
            et = ed_v[1, sl]
            seg = ed_v[0, sl] * R + et
            seg_v[sl] = seg
            ge_v[sl] = et * NP + ed_v[2, sl]
            hi_v[sl] = seg >> 7
            lo_v[sl] = seg & 127
        s1 = pltpu.async_copy(seg_v, seg_out.at[pl.ds(base, K)], sema)
        s2 = pltpu.async_copy(ge_v, ge_out.at[pl.ds(base, K)], semb)

        lane = jnp.arange(16, dtype=jnp.int32)

        def ohg(g, _):
            sl = pl.ds(g * 16, 16)
            lo16 = lo_v[sl]
            for l in range(16):
                e = g * 16 + l
                col = lo16[l]
                for j in range(8):
                    val = jnp.where(lane + (16 * j) == col, 1.0, 0.0)
                    oh_v[e, pl.ds(j * 16, 16)] = val
            return 0

        lax.fori_loop(0, K // 16, ohg, 0)
        s1.wait()
        s2.wait()
        pltpu.sync_copy(oh_v, cnt_sh.at[hi_v], add=True)
        return 0

    lax.fori_loop(0, EPW // K, chunk, 0)
    plsc.subcore_barrier()
    pltpu.sync_copy(cnt_sh.at[pl.ds(sid * zrows, zrows)],
                    counts_out.at[cid, pl.ds(sid * zrows, zrows)])


def _inv_body(c_ref, out_ref):
    out_ref[...] = 1.0 / jnp.maximum(c_ref[0] + c_ref[1], 1.0)


def _compute_inv(counts):
    return pl.pallas_call(
        _inv_body,
        grid=(CR // 128,),
        in_specs=[pl.BlockSpec((NC, 128, 128), lambda j: (0, j, 0))],
        out_specs=pl.BlockSpec((128, 128), lambda j: (j, 0)),
        out_shape=jax.ShapeDtypeStruct((CR, 128), jnp.float32),
    )(counts)


@functools.partial(
    pl.kernel,
    mesh=_mesh,
    out_type=jax.ShapeDtypeStruct((NC, NP, D), jnp.float32),
    scratch_types=[
        pltpu.VMEM((3, K), jnp.int32),
        pltpu.VMEM((K,), jnp.int32),
        pltpu.VMEM((K,), jnp.float32),
        pltpu.VMEM((K, 128), jnp.float32),
        pltpu.VMEM((K, D), jnp.float32),
        pltpu.VMEM((32,), jnp.float32),
        pltpu.VMEM_SHARED((NP, D), jnp.float32),
        pltpu.SemaphoreType.DMA,
        pltpu.SemaphoreType.DMA,
    ],
)
def _sc_edges(edata_hbm, ew_hbm, inv_hbm, h_hbm, zacc_hbm,
              acc_out,
              ed_v, hi_v, ew_v, inv_v, rows_v, fold_v, acc_sh,
              sem0, sem1):
    cid = lax.axis_index("c")
    sid = lax.axis_index("s")
    wid = sid * NC + cid
    zrows = NP // NS
    pltpu.sync_copy(zacc_hbm, acc_sh.at[pl.ds(sid * zrows, zrows)])
    plsc.subcore_barrier()
    lane = jnp.arange(16, dtype=jnp.int32)
    fold_v[pl.ds(16, 16)] = jnp.zeros((16,), jnp.float32)

    def chunk(k, _):
        base = wid * EPW + k * K
        pltpu.sync_copy(edata_hbm.at[wid * (EPW // K) + k], ed_v)
        for i in range(K // 16):
            sl = pl.ds(i * 16, 16)
            hi_v[sl] = ed_v[0, sl] >> 7
        c1 = pltpu.async_copy(inv_hbm.at[hi_v], inv_v, sem0)
        c2 = pltpu.async_copy(h_hbm.at[ed_v.at[2]], rows_v, sem1)
        pltpu.sync_copy(ew_hbm.at[pl.ds(base, K)], ew_v)
        c1.wait()
        c2.wait()

        def sgroup(g, _):
            sl = pl.ds(g * 16, 16)
            lo16 = ed_v[0, sl] & 127
            ew16 = ew_v[sl]
            for l in range(16):
                e = g * 16 + l
                col = lo16[l]
                acc = jnp.zeros((16,), jnp.float32)
                for j in range(8):
                    cs = pl.ds(j * 16, 16)
                    acc = acc + jnp.where(lane + (16 * j) == col,
                                          inv_v[e, cs], 0.0)
                fold_v[pl.ds(0, 16)] = acc
                for step in (8, 4, 2, 1):
                    t = fold_v[pl.ds(0, 16)] + fold_v[pl.ds(step, 16)]
                    fold_v[pl.ds(0, 16)] = t
                s = t[0] * ew16[l]
                for j in range(D // 16):
                    cs = pl.ds(j * 16, 16)
                    rows_v[e, cs] = rows_v[e, cs] * s
            return 0

        lax.fori_loop(0, K // 16, sgroup, 0)
        pltpu.sync_copy(rows_v, acc_sh.at[ed_v.at[1]], add=True)
        return 0

    lax.fori_loop(0, EPW // K, chunk, 0)
    plsc.subcore_barrier()
    pltpu.sync_copy(acc_sh.at[pl.ds(sid * zrows, zrows)],
                    acc_out.at[cid, pl.ds(sid * zrows, zrows)])


def _feat_body(acc_ref, x_ref, root_ref, bias_ref, out_ref):
    h = acc_ref[0] + acc_ref[1]
    h = h + jnp.dot(x_ref[...], root_ref[...],
                    preferred_element_type=jnp.float32)
    out_ref[...] = jnp.maximum(h + bias_ref[...], 0.0)


def _compute_features(acc, x_pad, root, bias2d):
    return pl.pallas_call(
        _feat_body,
        grid=(NP // TN,),
        in_specs=[
            pl.BlockSpec((NC, TN, D), lambda j: (0, j, 0)),
            pl.BlockSpec((TN, D), lambda j: (j, 0)),
            pl.BlockSpec((D, D), lambda j: (0, 0)),
            pl.BlockSpec((1, D), lambda j: (0, 0)),
        ],
        out_specs=pl.BlockSpec((TN, D), lambda j: (j, 0)),
        out_shape=jax.ShapeDtypeStruct((NP, D), jnp.float32),
    )(acc, x_pad, root, bias2d)


NID = BATCH * (2 + N_ASPECT)
IDW = NID // NW
IDC = 112


@functools.partial(
    pl.kernel,
    mesh=_mesh,
    out_type=jax.ShapeDtypeStruct((NID, D), jnp.float32),
    scratch_types=[
        pltpu.VMEM((IDC,), jnp.int32),
        pltpu.VMEM((IDC, D), jnp.float32),
        pltpu.SemaphoreType.DMA,
    ],
)
def _sc_gather_out(feat_hbm, ids_hbm, out_hbm, idx_v, rows_v, sem):
    cid = lax.axis_index("c")
    sid = lax.axis_index("s")
    wid = sid * NC + cid

    def chunk(k, _):
        base = wid * IDW + k * IDC
        pltpu.sync_copy(ids_hbm.at[pl.ds(base, IDC)], idx_v)
        pltpu.async_copy(feat_hbm.at[idx_v], rows_v, sem).wait()
        pltpu.sync_copy(rows_v, out_hbm.at[pl.ds(base, IDC)])
        return 0

    lax.fori_loop(0, IDW // IDC, chunk, 0)


def kernel(node_features, weight, root, bias, edge_weights, edge_index,
           edge_type, ent_user_ids, ent_item_ids, aspect_ent_ids):
    f32 = jnp.float32
    i32 = jnp.int32

    w5 = jnp.zeros((R, NB, DB, NB, DB), f32)
    bidx = jnp.arange(NB)
    w5 = w5.at[:, bidx, :, bidx, :].set(weight.transpose(1, 0, 2, 3))
    wd = w5.reshape(R, D, D)

    x_pad = jnp.zeros((NP, D), f32).at[:N_NODES].set(node_features)

    pad = EP - E
    agg = jnp.concatenate(
        [edge_index[0].astype(i32), jnp.full((pad,), NP - 1, i32)])
    mn = jnp.concatenate([edge_index[1].astype(i32), jnp.zeros((pad,), i32)])
    et = jnp.concatenate([edge_type.astype(i32), jnp.zeros((pad,), i32)])
    ew = jnp.concatenate([edge_weights[0].astype(f32), jnp.zeros((pad,), f32)])

    zeros_cnt = jnp.zeros((CR // NS, 128), f32)
    zeros_acc = jnp.zeros((NP // NS, D), f32)

    h = _compute_h(x_pad, wd).reshape(R * NP, D)
    edata_a = jnp.stack(
        [agg.reshape(EP // K, K), et.reshape(EP // K, K),
         mn.reshape(EP // K, K)], axis=1)
    counts, seg, ge = _sc_counts(edata_a, zeros_cnt)
    inv = _compute_inv(counts)
    edata_b = jnp.stack(
        [seg.reshape(EP // K, K), agg.reshape(EP // K, K),
         ge.reshape(EP // K, K)], axis=1)
    acc = _sc_edges(edata_b, ew, inv, h, zeros_acc)
    feats = _compute_features(acc, x_pad, root, bias.reshape(1, D))

    ids = jnp.concatenate([
        ent_user_ids.astype(i32),
        ent_item_ids.astype(i32),
        aspect_ent_ids.astype(i32).reshape(-1),
    ])
    g = _sc_gather_out(feats, ids)
    ent_user_rep = g[:BATCH][:, None, :]
    ent_item_rep = g[BATCH:2 * BATCH][:, None, :]
    ent_aspect_rep = g[2 * BATCH:].reshape(BATCH, N_ASPECT, D)
    return (ent_user_rep, ent_item_rep, ent_aspect_rep)


# speedup vs baseline: 2.1134x; 1.1768x over previous
import functools

import jax
import jax.numpy as jnp
from jax import lax
from jax.experimental import pallas as pl
from jax.experimental.pallas import tpu as pltpu
from jax.experimental.pallas import tpu_sc as plsc

N_NODES = 10002
D = 128
R = 16
NB = 8
DB = 16
E = 320000
BATCH = 1024
N_ASPECT = 5

NP = 10240
NRP = NP * R
CR = NRP // 128
NC = 2
NS = 16
NW = NC * NS
EP = 327680
EPW = EP // NW
K = 128
TN = 512

_mesh = plsc.VectorSubcoreMesh(core_axis_name="c", subcore_axis_name="s")


def _h_body(x_ref, w_ref, out_ref):
    out_ref[0] = jnp.dot(x_ref[...], w_ref[0],
                         preferred_element_type=jnp.float32)


def _compute_h(x_pad, wd):
    return pl.pallas_call(
        _h_body,
        grid=(R, NP // TN),
        in_specs=[
            pl.BlockSpec((TN, D), lambda r, j: (j, 0)),
            pl.BlockSpec((1, D, D), lambda r, j: (r, 0, 0)),
        ],
        out_specs=pl.BlockSpec((1, TN, D), lambda r, j: (r, j, 0)),
        out_shape=jax.ShapeDtypeStruct((R, NP, D), jnp.float32),
    )(x_pad, wd)


@functools.partial(
    pl.kernel,
    mesh=_mesh,
    out_type=(
        jax.ShapeDtypeStruct((NC, CR, 128), jnp.float32),
        jax.ShapeDtypeStruct((EP,), jnp.int32),
        jax.ShapeDtypeStruct((EP,), jnp.int32),
    ),
    scratch_types=[
        pltpu.VMEM((3, K), jnp.int32),
        pltpu.VMEM((K,), jnp.int32),
        pltpu.VMEM((K,), jnp.int32),
        pltpu.VMEM((K,), jnp.int32),
        pltpu.VMEM((K,), jnp.int32),
        pltpu.VMEM((K, 128), jnp.float32),
        pltpu.VMEM_SHARED((CR, 128), jnp.float32),
        pltpu.SemaphoreType.DMA,
        pltpu.SemaphoreType.DMA,
    ],
)
def _sc_counts(edata_hbm, zeros_hbm,
               counts_out, seg_out, ge_out,
               ed_v, seg_v, ge_v, hi_v, lo_v, oh_v, cnt_sh, sema, semb):
    cid = lax.axis_index("c")
    sid = lax.axis_index("s")
    wid = sid * NC + cid
    zrows = CR // NS
    pltpu.sync_copy(zeros_hbm, cnt_sh.at[pl.ds(sid * zrows, zrows)])
    plsc.subcore_barrier()

    def chunk(k, _):
        base = wid * EPW + k * K
        pltpu.sync_copy(edata_hbm.at[wid * (EPW // K) + k], ed_v)
        for i in range(K // 16):
            sl = pl.ds(i * 16, 16)
            et = ed_v[1, sl]
            seg = ed_v[0, sl] * R + et
            seg_v[sl] = seg
            ge_v[sl] = et * NP + ed_v[2, sl]
            hi_v[sl] = seg >> 7
            lo_v[sl] = seg & 127
        s1 = pltpu.async_copy(seg_v, seg_out.at[pl.ds(base, K)], sema)
        s2 = pltpu.async_copy(ge_v, ge_out.at[pl.ds(base, K)], semb)

        lane = jnp.arange(16, dtype=jnp.int32)

        def ohg(g, _):
            sl = pl.ds(g * 16, 16)
            lo16 = lo_v[sl]
            for l in range(16):
                e = g * 16 + l
                col = lo16[l]
                for j in range(8):
                    val = jnp.where(lane + (16 * j) == col, 1.0, 0.0)
                    oh_v[e, pl.ds(j * 16, 16)] = val
            return 0

        lax.fori_loop(0, K // 16, ohg, 0)
        s1.wait()
        s2.wait()
        pltpu.sync_copy(oh_v, cnt_sh.at[hi_v], add=True)
        return 0

    lax.fori_loop(0, EPW // K, chunk, 0)
    plsc.subcore_barrier()
    pltpu.sync_copy(cnt_sh.at[pl.ds(sid * zrows, zrows)],
                    counts_out.at[cid, pl.ds(sid * zrows, zrows)])


def _inv_body(c_ref, out_ref):
    out_ref[...] = 1.0 / jnp.maximum(c_ref[0] + c_ref[1], 1.0)


def _compute_inv(counts):
    return pl.pallas_call(
        _inv_body,
        grid=(CR // 128,),
        in_specs=[pl.BlockSpec((NC, 128, 128), lambda j: (0, j, 0))],
        out_specs=pl.BlockSpec((128, 128), lambda j: (j, 0)),
        out_shape=jax.ShapeDtypeStruct((CR, 128), jnp.float32),
    )(counts)


K2 = 64
NCH = EPW // K2


@functools.partial(
    pl.kernel,
    mesh=_mesh,
    out_type=jax.ShapeDtypeStruct((NC, NP, D), jnp.float32),
    scratch_types=[
        pltpu.VMEM((3, K2), jnp.int32),
        pltpu.VMEM((3, K2), jnp.int32),
        pltpu.VMEM((K2,), jnp.int32),
        pltpu.VMEM((K2,), jnp.int32),
        pltpu.VMEM((K2,), jnp.float32),
        pltpu.VMEM((K2,), jnp.float32),
        pltpu.VMEM((K2, 128), jnp.float32),
        pltpu.VMEM((K2, 128), jnp.float32),
        pltpu.VMEM((K2, D), jnp.float32),
        pltpu.VMEM((K2, D), jnp.float32),
        pltpu.VMEM((32,), jnp.float32),
        pltpu.VMEM_SHARED((NP, D), jnp.float32),
        pltpu.SemaphoreType.DMA,
        pltpu.SemaphoreType.DMA,
        pltpu.SemaphoreType.DMA,
        pltpu.SemaphoreType.DMA,
    ],
)
def _sc_edges(edata_hbm, ew_hbm, inv_hbm, h_hbm, zacc_hbm,
              acc_out,
              ed_a, ed_b, hi_a, hi_b, ew_a, ew_b, inv_a, inv_b,
              rows_a, rows_b, fold_v, acc_sh, sem0, sem1, sem2, sem3):
    cid = lax.axis_index("c")
    sid = lax.axis_index("s")
    wid = sid * NC + cid
    zrows = NP // NS
    pltpu.sync_copy(zacc_hbm, acc_sh.at[pl.ds(sid * zrows, zrows)])
    plsc.subcore_barrier()
    lane = jnp.arange(16, dtype=jnp.int32)
    fold_v[pl.ds(16, 16)] = jnp.zeros((16,), jnp.float32)

    def load_ed(ed_v, hi_v, ew_v, k):
        kc = jnp.minimum(wid * NCH + k, EP // K2 - 1)
        bc = jnp.minimum(wid * EPW + k * K2, EP - K2)
        pltpu.sync_copy(edata_hbm.at[kc], ed_v)
        pltpu.sync_copy(ew_hbm.at[pl.ds(bc, K2)], ew_v)
        for i in range(K2 // 16):
            sl = pl.ds(i * 16, 16)
            hi_v[sl] = ed_v[0, sl] >> 7

    def start_g(ed_v, hi_v, inv_v, rows_v, s0, s1):
        c1 = pltpu.async_copy(inv_hbm.at[hi_v], inv_v, s0)
        c2 = pltpu.async_copy(h_hbm.at[ed_v.at[2]], rows_v, s1)
        return c1, c2

    def compute(ed_v, ew_v, inv_v, rows_v):
        def sgroup(g, _):
            sl = pl.ds(g * 16, 16)
            lo16 = ed_v[0, sl] & 127
            ew16 = ew_v[sl]
            for l in range(16):
                e = g * 16 + l
                col = lo16[l]
                acc = jnp.zeros((16,), jnp.float32)
                for j in range(8):
                    cs = pl.ds(j * 16, 16)
                    acc = acc + jnp.where(lane + (16 * j) == col,
                                          inv_v[e, cs], 0.0)
                fold_v[pl.ds(0, 16)] = acc
                for step in (8, 4, 2, 1):
                    t = fold_v[pl.ds(0, 16)] + fold_v[pl.ds(step, 16)]
                    fold_v[pl.ds(0, 16)] = t
                s = t[0] * ew16[l]
                for j in range(D // 16):
                    cs = pl.ds(j * 16, 16)
                    rows_v[e, cs] = rows_v[e, cs] * s
            return 0

        lax.fori_loop(0, K2 // 16, sgroup, 0)
        pltpu.sync_copy(rows_v, acc_sh.at[ed_v.at[1]], add=True)

    load_ed(ed_a, hi_a, ew_a, 0)

    def pair(kp, _):
        k0 = 2 * kp
        a1, a2 = start_g(ed_a, hi_a, inv_a, rows_a, sem0, sem1)
        load_ed(ed_b, hi_b, ew_b, k0 + 1)
        b1, b2 = start_g(ed_b, hi_b, inv_b, rows_b, sem2, sem3)
        a1.wait()
        a2.wait()
        compute(ed_a, ew_a, inv_a, rows_a)
        load_ed(ed_a, hi_a, ew_a, k0 + 2)
        b1.wait()
        b2.wait()
        compute(ed_b, ew_b, inv_b, rows_b)
        return 0

    lax.fori_loop(0, NCH // 2, pair, 0)
    plsc.subcore_barrier()
    pltpu.sync_copy(acc_sh.at[pl.ds(sid * zrows, zrows)],
                    acc_out.at[cid, pl.ds(sid * zrows, zrows)])


def _feat_body(acc_ref, x_ref, root_ref, bias_ref, out_ref):
    h = acc_ref[0] + acc_ref[1]
    h = h + jnp.dot(x_ref[...], root_ref[...],
                    preferred_element_type=jnp.float32)
    out_ref[...] = jnp.maximum(h + bias_ref[...], 0.0)


def _compute_features(acc, x_pad, root, bias2d):
    return pl.pallas_call(
        _feat_body,
        grid=(NP // TN,),
        in_specs=[
            pl.BlockSpec((NC, TN, D), lambda j: (0, j, 0)),
            pl.BlockSpec((TN, D), lambda j: (j, 0)),
            pl.BlockSpec((D, D), lambda j: (0, 0)),
            pl.BlockSpec((1, D), lambda j: (0, 0)),
        ],
        out_specs=pl.BlockSpec((TN, D), lambda j: (j, 0)),
        out_shape=jax.ShapeDtypeStruct((NP, D), jnp.float32),
    )(acc, x_pad, root, bias2d)


NID = BATCH * (2 + N_ASPECT)
IDW = NID // NW
IDC = 112


@functools.partial(
    pl.kernel,
    mesh=_mesh,
    out_type=jax.ShapeDtypeStruct((NID, D), jnp.float32),
    scratch_types=[
        pltpu.VMEM((IDC,), jnp.int32),
        pltpu.VMEM((IDC, D), jnp.float32),
        pltpu.SemaphoreType.DMA,
    ],
)
def _sc_gather_out(feat_hbm, ids_hbm, out_hbm, idx_v, rows_v, sem):
    cid = lax.axis_index("c")
    sid = lax.axis_index("s")
    wid = sid * NC + cid

    def chunk(k, _):
        base = wid * IDW + k * IDC
        pltpu.sync_copy(ids_hbm.at[pl.ds(base, IDC)], idx_v)
        pltpu.async_copy(feat_hbm.at[idx_v], rows_v, sem).wait()
        pltpu.sync_copy(rows_v, out_hbm.at[pl.ds(base, IDC)])
        return 0

    lax.fori_loop(0, IDW // IDC, chunk, 0)


def kernel(node_features, weight, root, bias, edge_weights, edge_index,
           edge_type, ent_user_ids, ent_item_ids, aspect_ent_ids):
    f32 = jnp.float32
    i32 = jnp.int32

    w5 = jnp.zeros((R, NB, DB, NB, DB), f32)
    bidx = jnp.arange(NB)
    w5 = w5.at[:, bidx, :, bidx, :].set(weight.transpose(1, 0, 2, 3))
    wd = w5.reshape(R, D, D)

    x_pad = jnp.zeros((NP, D), f32).at[:N_NODES].set(node_features)

    pad = EP - E
    agg = jnp.concatenate(
        [edge_index[0].astype(i32), jnp.full((pad,), NP - 1, i32)])
    mn = jnp.concatenate([edge_index[1].astype(i32), jnp.zeros((pad,), i32)])
    et = jnp.concatenate([edge_type.astype(i32), jnp.zeros((pad,), i32)])
    ew = jnp.concatenate([edge_weights[0].astype(f32), jnp.zeros((pad,), f32)])

    zeros_cnt = jnp.zeros((CR // NS, 128), f32)
    zeros_acc = jnp.zeros((NP // NS, D), f32)

    h = _compute_h(x_pad, wd).reshape(R * NP, D)
    edata_a = jnp.stack(
        [agg.reshape(EP // K, K), et.reshape(EP // K, K),
         mn.reshape(EP // K, K)], axis=1)
    counts, seg, ge = _sc_counts(edata_a, zeros_cnt)
    inv = _compute_inv(counts)
    edata_b = jnp.stack(
        [seg.reshape(EP // 64, 64), agg.reshape(EP // 64, 64),
         ge.reshape(EP // 64, 64)], axis=1)
    acc = _sc_edges(edata_b, ew, inv, h, zeros_acc)
    feats = _compute_features(acc, x_pad, root, bias.reshape(1, D))

    ids = jnp.concatenate([
        ent_user_ids.astype(i32),
        ent_item_ids.astype(i32),
        aspect_ent_ids.astype(i32).reshape(-1),
    ])
    g = _sc_gather_out(feats, ids)
    ent_user_rep = g[:BATCH][:, None, :]
    ent_item_rep = g[BATCH:2 * BATCH][:, None, :]
    ent_aspect_rep = g[2 * BATCH:].reshape(BATCH, N_ASPECT, D)
    return (ent_user_rep, ent_item_rep, ent_aspect_rep)
